# Initial kernel scaffold; baseline (speedup 1.0000x reference)
import jax
import jax.numpy as jnp
from jax.experimental import pallas as pl


def kernel(x, solutions, visited_time, W1, b1, W2, b2, pattern):
    raise NotImplementedError("write your pallas kernel here")


# speedup vs baseline: 3.1053x; 3.1053x over previous
import functools

import jax
import jax.numpy as jnp
from jax import lax
from jax.experimental import pallas as pl
from jax.experimental.pallas import tpu as pltpu
from jax.experimental.pallas import tpu_sc as plsc

EMB = 128
HID = 64

_NC = 2
_NS = 16
_NW = _NC * _NS


def _mlp_body(x_ref, w1_ref, b1_ref, w2_ref, b2_ref, o_ref):
    x = x_ref[...]
    w1 = w1_ref[...]
    h = x[:, 0:1] * w1[0:1, :] + x[:, 1:2] * w1[1:2, :] + b1_ref[...]
    h = jnp.maximum(h, 0.0)
    o_ref[...] = (
        jax.lax.dot_general(h, w2_ref[...], (((1,), (0,)), ((), ())),
                            preferred_element_type=jnp.float32)
        + b2_ref[...]
    )


def _mlp(xf, W1, b1, W2, b2, rows_per_block=2048):
    n = xf.shape[0]
    grid = n // rows_per_block
    return pl.pallas_call(
        _mlp_body,
        grid=(grid,),
        in_specs=[
            pl.BlockSpec((rows_per_block, xf.shape[1]), lambda i: (i, 0)),
            pl.BlockSpec((2, HID), lambda i: (0, 0)),
            pl.BlockSpec((1, HID), lambda i: (0, 0)),
            pl.BlockSpec((HID, EMB), lambda i: (0, 0)),
            pl.BlockSpec((1, EMB), lambda i: (0, 0)),
        ],
        out_specs=pl.BlockSpec((rows_per_block, EMB), lambda i: (i, 0)),
        out_shape=jax.ShapeDtypeStruct((n, EMB), jnp.float32),
    )(xf, W1, b1.reshape(1, HID), W2, b2.reshape(1, EMB))


def _make_gather(total, chunk=128):
    per_w = total // _NW
    n_ch = per_w // chunk
    mesh = plsc.VectorSubcoreMesh(core_axis_name="c", subcore_axis_name="s")

    @functools.partial(
        pl.kernel,
        out_type=jax.ShapeDtypeStruct((total, EMB), jnp.float32),
        mesh=mesh,
        scratch_types=[
            pltpu.VMEM((per_w,), jnp.int32),
            pltpu.VMEM((chunk, EMB), jnp.float32),
            pltpu.SemaphoreType.DMA,
        ],
    )
    def gather_k(idx_hbm, table_hbm, out_hbm, idx_v, buf, sem):
        wid = lax.axis_index("s") * _NC + lax.axis_index("c")
        base = wid * per_w
        pltpu.sync_copy(idx_hbm.at[pl.ds(base, per_w)], idx_v)

        def body(i, carry):
            off = i * chunk
            pltpu.async_copy(table_hbm.at[idx_v.at[pl.ds(off, chunk)]],
                             buf, sem).wait()
            pltpu.sync_copy(buf, out_hbm.at[pl.ds(base + off, chunk)])
            return carry

        lax.fori_loop(0, n_ch, body, 0)

    return gather_k


def kernel(x, solutions, visited_time, W1, b1, W2, b2, pattern):
    bs, seq, nd = x.shape
    total = bs * seq
    idx = visited_time.reshape(total).astype(jnp.int32)
    emb = _mlp(x.reshape(total, nd), W1, b1, W2, b2).reshape(bs, seq, EMB)
    pos = _make_gather(total)(idx, pattern).reshape(bs, seq, EMB)
    return (emb, pos)


# speedup vs baseline: 4.7515x; 1.5301x over previous
import functools

import jax
import jax.numpy as jnp
from jax import lax
from jax.experimental import pallas as pl
from jax.experimental.pallas import tpu as pltpu
from jax.experimental.pallas import tpu_sc as plsc

EMB = 128
HID = 64

_NC = 2
_NS = 16
_NW = _NC * _NS


def _mlp_body(nsub, a0_ref, a1_ref, w10_ref, w11_ref, b1_ref, w2_ref,
              b2_ref, o_ref):
    w10 = w10_ref[...]
    w11 = w11_ref[...]
    b1 = b1_ref[...]
    w2 = w2_ref[...]
    b2 = b2_ref[...]
    for s in range(nsub):
        c0 = a0_ref[0, :, s:s + 1]
        c1 = a1_ref[0, :, s:s + 1]
        h = jnp.maximum(c0 * w10 + c1 * w11 + b1, 0.0)
        o_ref[s * 128:(s + 1) * 128, :] = (
            jax.lax.dot_general(h, w2, (((1,), (0,)), ((), ())),
                                preferred_element_type=jnp.float32)
            + b2
        )


def _mlp(a0, a1, W1, b1, W2, b2):
    grid, _, nsub = a0.shape
    rows_per_block = nsub * 128
    n = grid * rows_per_block
    return pl.pallas_call(
        functools.partial(_mlp_body, nsub),
        grid=(grid,),
        in_specs=[
            pl.BlockSpec((1, 128, nsub), lambda i: (i, 0, 0)),
            pl.BlockSpec((1, 128, nsub), lambda i: (i, 0, 0)),
            pl.BlockSpec((1, HID), lambda i: (0, 0)),
            pl.BlockSpec((1, HID), lambda i: (0, 0)),
            pl.BlockSpec((1, HID), lambda i: (0, 0)),
            pl.BlockSpec((HID, EMB), lambda i: (0, 0)),
            pl.BlockSpec((1, EMB), lambda i: (0, 0)),
        ],
        out_specs=pl.BlockSpec((rows_per_block, EMB), lambda i: (i, 0)),
        out_shape=jax.ShapeDtypeStruct((n, EMB), jnp.float32),
    )(a0, a1, W1[0:1, :], W1[1:2, :], b1.reshape(1, HID), W2,
      b2.reshape(1, EMB))


def _make_gather(total, chunk=128):
    per_w = total // _NW
    n_ch = per_w // chunk
    mesh = plsc.VectorSubcoreMesh(core_axis_name="c", subcore_axis_name="s")

    @functools.partial(
        pl.kernel,
        out_type=jax.ShapeDtypeStruct((total, EMB), jnp.float32),
        mesh=mesh,
        scratch_types=[
            pltpu.VMEM((per_w,), jnp.int32),
            pltpu.VMEM((chunk, EMB), jnp.float32),
            pltpu.SemaphoreType.DMA,
        ],
    )
    def gather_k(idx_hbm, table_hbm, out_hbm, idx_v, buf, sem):
        wid = lax.axis_index("s") * _NC + lax.axis_index("c")
        base = wid * per_w
        pltpu.sync_copy(idx_hbm.at[pl.ds(base, per_w)], idx_v)

        def body(i, carry):
            off = i * chunk
            pltpu.async_copy(table_hbm.at[idx_v.at[pl.ds(off, chunk)]],
                             buf, sem).wait()
            pltpu.sync_copy(buf, out_hbm.at[pl.ds(base + off, chunk)])
            return carry

        lax.fori_loop(0, n_ch, body, 0)

    return gather_k


def kernel(x, solutions, visited_time, W1, b1, W2, b2, pattern):
    bs, seq, nd = x.shape
    total = bs * seq
    idx = visited_time.reshape(total).astype(jnp.int32)
    rows_per_block = 2048
    grid = total // rows_per_block
    nsub = rows_per_block // 128
    a0 = x[:, :, 0].reshape(grid, nsub, 128).transpose(0, 2, 1)
    a1 = x[:, :, 1].reshape(grid, nsub, 128).transpose(0, 2, 1)
    emb = _mlp(a0, a1, W1, b1, W2, b2).reshape(bs, seq, EMB)
    pos = _make_gather(total)(idx, pattern).reshape(bs, seq, EMB)
    return (emb, pos)
